# Initial kernel scaffold; baseline (speedup 1.0000x reference)
import jax
import jax.numpy as jnp
from jax.experimental import pallas as pl


def kernel(x, edge_index, W1, b1, W2, b2):
    raise NotImplementedError("write your pallas kernel here")


# speedup vs baseline: 33.3124x; 33.3124x over previous
import functools

import jax
import jax.numpy as jnp
from jax import lax
from jax.experimental import pallas as pl
from jax.experimental.pallas import tpu as pltpu
from jax.experimental.pallas import tpu_sc as plsc

N = 10000
E = 320000
D_IN = 128
D_H = 16

NC = 2
NS = 16
NW = NC * NS
CH = 128
NCHUNK = -(-E // (NW * CH))
EPW = NCHUNK * CH
E_PAD = NW * EPW
N_PAD = 10112
RT = N_PAD // NS

@functools.cache
def _mesh():
    return plsc.VectorSubcoreMesh(
        core_axis_name="c", subcore_axis_name="s",
        num_cores=NC, num_subcores=NS)


def _sc_deg_body(dst3, zeros1, deg_out, dst_v, ones_v, buf_v, sh_deg):
    c = lax.axis_index("c")
    s = lax.axis_index("s")
    w = c * NS + s
    pltpu.sync_copy(zeros1.at[pl.ds(s * RT, RT)], buf_v)
    pltpu.sync_copy(buf_v, sh_deg.at[pl.ds(s * RT, RT)])
    for i in range(CH // 16):
        ones_v[pl.ds(i * 16, 16)] = jnp.ones((16,), jnp.float32)
    pltpu.sync_copy(dst3.at[w], dst_v)
    plsc.subcore_barrier()

    def body(j, carry):
        pltpu.sync_copy(ones_v, sh_deg.at[dst_v.at[j]], add=True)
        return carry

    lax.fori_loop(0, NCHUNK, body, 0)
    plsc.subcore_barrier()
    off = pl.multiple_of(c * N_PAD + s * RT, 8)
    pltpu.sync_copy(sh_deg.at[pl.ds(s * RT, RT)], buf_v)
    pltpu.sync_copy(buf_v, deg_out.at[pl.ds(off, RT)])


@functools.cache
def _sc_deg():
    return pl.kernel(
        _sc_deg_body,
        out_type=jax.ShapeDtypeStruct((NC * N_PAD,), jnp.float32),
        mesh=_mesh(),
        scratch_types=[
            pltpu.VMEM((NCHUNK, CH), jnp.int32),
            pltpu.VMEM((CH,), jnp.float32),
            pltpu.VMEM((RT,), jnp.float32),
            pltpu.VMEM_SHARED((N_PAD,), jnp.float32),
        ],
    )


def _sc_agg_body(hp, src3, dst3, zeros2, agg_out, src_v, dst_v, rows_v,
                 buf_v, sh_agg, sem):
    c = lax.axis_index("c")
    s = lax.axis_index("s")
    w = c * NS + s
    pltpu.sync_copy(zeros2.at[pl.ds(s * RT, RT)], buf_v)
    pltpu.sync_copy(buf_v, sh_agg.at[pl.ds(s * RT, RT)])
    pltpu.sync_copy(src3.at[w], src_v)
    pltpu.sync_copy(dst3.at[w], dst_v)
    plsc.subcore_barrier()

    def body(j, carry):
        pltpu.async_copy(hp.at[src_v.at[j]], rows_v, sem).wait()
        pltpu.sync_copy(rows_v, sh_agg.at[dst_v.at[j]], add=True)
        return carry

    lax.fori_loop(0, NCHUNK, body, 0)
    plsc.subcore_barrier()
    pltpu.sync_copy(sh_agg.at[pl.ds(s * RT, RT)], buf_v)
    pltpu.sync_copy(buf_v, agg_out.at[c, pl.ds(s * RT, RT)])


@functools.cache
def _sc_agg():
    return pl.kernel(
        _sc_agg_body,
        out_type=jax.ShapeDtypeStruct((NC, N_PAD, D_H), jnp.float32),
        mesh=_mesh(),
        scratch_types=[
            pltpu.VMEM((NCHUNK, CH), jnp.int32),
            pltpu.VMEM((NCHUNK, CH), jnp.int32),
            pltpu.VMEM((CH, D_H), jnp.float32),
            pltpu.VMEM((RT, D_H), jnp.float32),
            pltpu.VMEM_SHARED((N_PAD, D_H), jnp.float32),
            pltpu.SemaphoreType.DMA,
        ],
        compiler_params=pltpu.CompilerParams(use_tc_tiling_on_sc=False),
    )


_BM = 1264


def _tc_a_body(x_ref, degT_ref, w1_ref, hp_ref, dinv_ref):
    deg = degT_ref[...]
    d = deg[:, 0:1] + deg[:, 1:2] + 1.0
    dinv = lax.rsqrt(d)
    h = jnp.dot(x_ref[...], w1_ref[...], preferred_element_type=jnp.float32)
    hp_ref[...] = h * dinv
    dinv_ref[...] = dinv


_tc_a = pl.pallas_call(
    _tc_a_body,
    grid=(N_PAD // _BM,),
    in_specs=[
        pl.BlockSpec((_BM, D_IN), lambda i: (i, 0)),
        pl.BlockSpec((_BM, NC), lambda i: (i, 0)),
        pl.BlockSpec((D_IN, D_H), lambda i: (0, 0)),
    ],
    out_specs=[
        pl.BlockSpec((_BM, D_H), lambda i: (i, 0)),
        pl.BlockSpec((_BM, 1), lambda i: (i, 0)),
    ],
    out_shape=[
        jax.ShapeDtypeStruct((N_PAD, D_H), jnp.float32),
        jax.ShapeDtypeStruct((N_PAD, 1), jnp.float32),
    ],
)


def _tc_b_body(agg_ref, hp_ref, dinv_ref, b1_ref, w2_ref, out_ref):
    a3 = agg_ref[...]
    a = a3[0] + a3[1] + hp_ref[...]
    dinv = dinv_ref[...]
    z = jnp.maximum(a * dinv + b1_ref[...], 0.0)
    out_ref[...] = jnp.dot(
        z, w2_ref[...], preferred_element_type=jnp.float32) * dinv


_tc_b = pl.pallas_call(
    _tc_b_body,
    grid=(N_PAD // _BM,),
    in_specs=[
        pl.BlockSpec((NC, _BM, D_H), lambda i: (0, i, 0)),
        pl.BlockSpec((_BM, D_H), lambda i: (i, 0)),
        pl.BlockSpec((_BM, 1), lambda i: (i, 0)),
        pl.BlockSpec((1, D_H), lambda i: (0, 0)),
        pl.BlockSpec((D_H, D_H), lambda i: (0, 0)),
    ],
    out_specs=pl.BlockSpec((_BM, D_H), lambda i: (i, 0)),
    out_shape=jax.ShapeDtypeStruct((N_PAD, D_H), jnp.float32),
)


def _tc_c_body(agg_ref, hp_ref, dinv_ref, b2_ref, out_ref):
    a3 = agg_ref[...]
    a = a3[0] + a3[1] + hp_ref[...]
    out_ref[...] = a * dinv_ref[...] + b2_ref[...]


_tc_c = pl.pallas_call(
    _tc_c_body,
    grid=(N_PAD // _BM,),
    in_specs=[
        pl.BlockSpec((NC, _BM, D_H), lambda i: (0, i, 0)),
        pl.BlockSpec((_BM, D_H), lambda i: (i, 0)),
        pl.BlockSpec((_BM, 1), lambda i: (i, 0)),
        pl.BlockSpec((1, D_H), lambda i: (0, 0)),
    ],
    out_specs=pl.BlockSpec((_BM, D_H), lambda i: (i, 0)),
    out_shape=jax.ShapeDtypeStruct((N_PAD, D_H), jnp.float32),
)


@jax.jit
def kernel(x, edge_index, W1, b1, W2, b2):
    src = edge_index[0]
    dst = edge_index[1]
    pad = jnp.full((E_PAD - E,), N, jnp.int32)
    src3 = jnp.concatenate([src, pad]).reshape(NW, NCHUNK, CH)
    dst3 = jnp.concatenate([dst, pad]).reshape(NW, NCHUNK, CH)
    zeros1 = jnp.zeros((N_PAD,), jnp.float32)
    zeros2 = jnp.zeros((N_PAD, D_H), jnp.float32)
    x_pad = jnp.concatenate([x, jnp.zeros((N_PAD - N, D_IN), jnp.float32)])

    deg2 = _sc_deg()(dst3, zeros1)
    degT = deg2.reshape(NC, N_PAD).T
    hp1, dinv = _tc_a(x_pad, degT, W1)
    agg1 = _sc_agg()(hp1, src3, dst3, zeros2)
    hp2 = _tc_b(agg1, hp1, dinv, b1.reshape(1, D_H), W2)
    agg2 = _sc_agg()(hp2, src3, dst3, zeros2)
    out_pad = _tc_c(agg2, hp2, dinv, b2.reshape(1, D_H))
    return out_pad[:N]


# speedup vs baseline: 39.0482x; 1.1722x over previous
import functools

import jax
import jax.numpy as jnp
from jax import lax
from jax.experimental import pallas as pl
from jax.experimental.pallas import tpu as pltpu
from jax.experimental.pallas import tpu_sc as plsc

N = 10000
E = 320000
D_IN = 128
D_H = 16

NC = 2
NS = 16
NW = NC * NS
CH = 128
NCHUNK = 80
EPW = NCHUNK * CH
E_PAD = NW * EPW
N_PAD = 10112
RT = N_PAD // NS

@functools.cache
def _mesh():
    return plsc.VectorSubcoreMesh(
        core_axis_name="c", subcore_axis_name="s",
        num_cores=NC, num_subcores=NS)


def _sc_deg_body(dst3, zeros1, deg_out, dst_v, ones_v, buf_v, sh_deg, sem):
    c = lax.axis_index("c")
    s = lax.axis_index("s")
    w = c * NS + s
    pltpu.sync_copy(zeros1.at[pl.ds(s * RT, RT)], buf_v)
    pltpu.sync_copy(buf_v, sh_deg.at[pl.ds(s * RT, RT)])
    for i in range(CH // 16):
        ones_v[pl.ds(i * 16, 16)] = jnp.ones((16,), jnp.float32)
    pltpu.sync_copy(dst3.at[w], dst_v)
    plsc.subcore_barrier()

    descs = [pltpu.async_copy(ones_v, sh_deg.at[dst_v.at[j]], sem, add=True)
             for j in range(NCHUNK)]
    for d in descs:
        d.wait()
    plsc.subcore_barrier()
    off = pl.multiple_of(c * N_PAD + s * RT, 8)
    pltpu.sync_copy(sh_deg.at[pl.ds(s * RT, RT)], buf_v)
    pltpu.sync_copy(buf_v, deg_out.at[pl.ds(off, RT)])


@functools.cache
def _sc_deg():
    return pl.kernel(
        _sc_deg_body,
        out_type=jax.ShapeDtypeStruct((NC * N_PAD,), jnp.float32),
        mesh=_mesh(),
        scratch_types=[
            pltpu.VMEM((NCHUNK, CH), jnp.int32),
            pltpu.VMEM((CH,), jnp.float32),
            pltpu.VMEM((RT,), jnp.float32),
            pltpu.VMEM_SHARED((N_PAD,), jnp.float32),
            pltpu.SemaphoreType.DMA,
        ],
    )


_NBUF = 4


def _sc_agg_body(hp, src3, dst3, zeros2, agg_out, src_v, dst_v, r0, r1, r2,
                 r3, buf_v, sh_agg, s0, s1, s2, s3):
    c = lax.axis_index("c")
    s = lax.axis_index("s")
    w = c * NS + s
    rows = [r0, r1, r2, r3]
    sems = [s0, s1, s2, s3]
    pltpu.sync_copy(zeros2.at[pl.ds(s * RT, RT)], buf_v)
    pltpu.sync_copy(buf_v, sh_agg.at[pl.ds(s * RT, RT)])
    pltpu.sync_copy(src3.at[w], src_v)
    pltpu.sync_copy(dst3.at[w], dst_v)
    plsc.subcore_barrier()

    descs = [None] * NCHUNK
    for j in range(_NBUF):
        descs[j] = pltpu.async_copy(hp.at[src_v.at[j]], rows[j], sems[j])
    for j in range(NCHUNK):
        b = j % _NBUF
        descs[j].wait()
        pltpu.sync_copy(rows[b], sh_agg.at[dst_v.at[j]], add=True)
        if j + _NBUF < NCHUNK:
            descs[j + _NBUF] = pltpu.async_copy(
                hp.at[src_v.at[j + _NBUF]], rows[b], sems[b])
    plsc.subcore_barrier()
    pltpu.sync_copy(sh_agg.at[pl.ds(s * RT, RT)], buf_v)
    pltpu.sync_copy(buf_v, agg_out.at[c, pl.ds(s * RT, RT)])


@functools.cache
def _sc_agg():
    return pl.kernel(
        _sc_agg_body,
        out_type=jax.ShapeDtypeStruct((NC, N_PAD, D_H), jnp.float32),
        mesh=_mesh(),
        scratch_types=[
            pltpu.VMEM((NCHUNK, CH), jnp.int32),
            pltpu.VMEM((NCHUNK, CH), jnp.int32),
            pltpu.VMEM((CH, D_H), jnp.float32),
            pltpu.VMEM((CH, D_H), jnp.float32),
            pltpu.VMEM((CH, D_H), jnp.float32),
            pltpu.VMEM((CH, D_H), jnp.float32),
            pltpu.VMEM((RT, D_H), jnp.float32),
            pltpu.VMEM_SHARED((N_PAD, D_H), jnp.float32),
            pltpu.SemaphoreType.DMA,
            pltpu.SemaphoreType.DMA,
            pltpu.SemaphoreType.DMA,
            pltpu.SemaphoreType.DMA,
        ],
        compiler_params=pltpu.CompilerParams(use_tc_tiling_on_sc=False),
    )


_BM = 1264


def _tc_a_body(x_ref, degT_ref, w1_ref, hp_ref, dinv_ref):
    deg = degT_ref[...]
    d = deg[:, 0:1] + deg[:, 1:2] + 1.0
    dinv = lax.rsqrt(d)
    h = jnp.dot(x_ref[...], w1_ref[...], preferred_element_type=jnp.float32)
    hp_ref[...] = h * dinv
    dinv_ref[...] = dinv


_tc_a = pl.pallas_call(
    _tc_a_body,
    grid=(N_PAD // _BM,),
    in_specs=[
        pl.BlockSpec((_BM, D_IN), lambda i: (i, 0)),
        pl.BlockSpec((_BM, NC), lambda i: (i, 0)),
        pl.BlockSpec((D_IN, D_H), lambda i: (0, 0)),
    ],
    out_specs=[
        pl.BlockSpec((_BM, D_H), lambda i: (i, 0)),
        pl.BlockSpec((_BM, 1), lambda i: (i, 0)),
    ],
    out_shape=[
        jax.ShapeDtypeStruct((N_PAD, D_H), jnp.float32),
        jax.ShapeDtypeStruct((N_PAD, 1), jnp.float32),
    ],
)


def _tc_b_body(agg_ref, hp_ref, dinv_ref, b1_ref, w2_ref, out_ref):
    a3 = agg_ref[...]
    a = a3[0] + a3[1] + hp_ref[...]
    dinv = dinv_ref[...]
    z = jnp.maximum(a * dinv + b1_ref[...], 0.0)
    out_ref[...] = jnp.dot(
        z, w2_ref[...], preferred_element_type=jnp.float32) * dinv


_tc_b = pl.pallas_call(
    _tc_b_body,
    grid=(N_PAD // _BM,),
    in_specs=[
        pl.BlockSpec((NC, _BM, D_H), lambda i: (0, i, 0)),
        pl.BlockSpec((_BM, D_H), lambda i: (i, 0)),
        pl.BlockSpec((_BM, 1), lambda i: (i, 0)),
        pl.BlockSpec((1, D_H), lambda i: (0, 0)),
        pl.BlockSpec((D_H, D_H), lambda i: (0, 0)),
    ],
    out_specs=pl.BlockSpec((_BM, D_H), lambda i: (i, 0)),
    out_shape=jax.ShapeDtypeStruct((N_PAD, D_H), jnp.float32),
)


def _tc_c_body(agg_ref, hp_ref, dinv_ref, b2_ref, out_ref):
    a3 = agg_ref[...]
    a = a3[0] + a3[1] + hp_ref[...]
    out_ref[...] = a * dinv_ref[...] + b2_ref[...]


_tc_c = pl.pallas_call(
    _tc_c_body,
    grid=(N_PAD // _BM,),
    in_specs=[
        pl.BlockSpec((NC, _BM, D_H), lambda i: (0, i, 0)),
        pl.BlockSpec((_BM, D_H), lambda i: (i, 0)),
        pl.BlockSpec((_BM, 1), lambda i: (i, 0)),
        pl.BlockSpec((1, D_H), lambda i: (0, 0)),
    ],
    out_specs=pl.BlockSpec((_BM, D_H), lambda i: (i, 0)),
    out_shape=jax.ShapeDtypeStruct((N_PAD, D_H), jnp.float32),
)


@jax.jit
def kernel(x, edge_index, W1, b1, W2, b2):
    src = edge_index[0]
    dst = edge_index[1]
    pad = jnp.full((E_PAD - E,), N, jnp.int32)
    src3 = jnp.concatenate([src, pad]).reshape(NW, NCHUNK, CH)
    dst3 = jnp.concatenate([dst, pad]).reshape(NW, NCHUNK, CH)
    zeros1 = jnp.zeros((N_PAD,), jnp.float32)
    zeros2 = jnp.zeros((N_PAD, D_H), jnp.float32)
    x_pad = jnp.concatenate([x, jnp.zeros((N_PAD - N, D_IN), jnp.float32)])

    deg2 = _sc_deg()(dst3, zeros1)
    degT = deg2.reshape(NC, N_PAD).T
    hp1, dinv = _tc_a(x_pad, degT, W1)
    agg1 = _sc_agg()(hp1, src3, dst3, zeros2)
    hp2 = _tc_b(agg1, hp1, dinv, b1.reshape(1, D_H), W2)
    agg2 = _sc_agg()(hp2, src3, dst3, zeros2)
    out_pad = _tc_c(agg2, hp2, dinv, b2.reshape(1, D_H))
    return out_pad[:N]


# speedup vs baseline: 72.4214x; 1.8547x over previous
import functools

import jax
import jax.numpy as jnp
from jax import lax
from jax.experimental import pallas as pl
from jax.experimental.pallas import tpu as pltpu
from jax.experimental.pallas import tpu_sc as plsc

N = 10000
E = 320000
D_IN = 128
D_H = 16

NC = 2
NS = 16
NW = NC * NS
CH = 125
NCHUNK = 80
EPW = NCHUNK * CH
RT = N // NS
ND = 10112
RTD = ND // NS
PK = N // 8


@functools.cache
def _mesh():
    return plsc.VectorSubcoreMesh(
        core_axis_name="c", subcore_axis_name="s",
        num_cores=NC, num_subcores=NS)


def _sc_deg_body(dst3, zeros1, deg_out, dst_v, ones_v, buf_v, sh_deg, sem):
    c = lax.axis_index("c")
    s = lax.axis_index("s")
    w = c * NS + s
    pltpu.sync_copy(zeros1.at[pl.ds(s * RTD, RTD)], buf_v)
    pltpu.sync_copy(buf_v, sh_deg.at[pl.ds(s * RTD, RTD)])
    for i in range(8):
        ones_v[pl.ds(i * 16, 16)] = jnp.ones((16,), jnp.float32)
    pltpu.sync_copy(dst3.at[w], dst_v)
    plsc.subcore_barrier()

    descs = [pltpu.async_copy(ones_v.at[pl.ds(0, CH)],
                              sh_deg.at[dst_v.at[j]], sem, add=True)
             for j in range(NCHUNK)]
    for d in descs:
        d.wait()
    plsc.subcore_barrier()
    off = pl.multiple_of(c * ND + s * RTD, 8)
    pltpu.sync_copy(sh_deg.at[pl.ds(s * RTD, RTD)], buf_v)
    pltpu.sync_copy(buf_v, deg_out.at[pl.ds(off, RTD)])


@functools.cache
def _sc_deg():
    return pl.kernel(
        _sc_deg_body,
        out_type=jax.ShapeDtypeStruct((NC * ND,), jnp.float32),
        mesh=_mesh(),
        scratch_types=[
            pltpu.VMEM((NCHUNK, CH), jnp.int32),
            pltpu.VMEM((128,), jnp.float32),
            pltpu.VMEM((RTD,), jnp.float32),
            pltpu.VMEM_SHARED((ND,), jnp.float32),
            pltpu.SemaphoreType.DMA,
        ],
        compiler_params=pltpu.CompilerParams(use_tc_tiling_on_sc=False),
    )


_NBUF = 4


def _sc_agg_body(hp, src3, dst3, zeros2, agg_out, src_v, dst_v, r0, r1, r2,
                 r3, buf_v, sh_agg, s0, s1, s2, s3):
    c = lax.axis_index("c")
    s = lax.axis_index("s")
    w = c * NS + s
    rows = [r0, r1, r2, r3]
    sems = [s0, s1, s2, s3]
    pltpu.sync_copy(zeros2.at[pl.ds(s * RT, RT)], buf_v)
    pltpu.sync_copy(buf_v, sh_agg.at[pl.ds(s * RT, RT)])
    pltpu.sync_copy(src3.at[w], src_v)
    pltpu.sync_copy(dst3.at[w], dst_v)
    plsc.subcore_barrier()

    descs = [None] * NCHUNK
    for j in range(_NBUF):
        descs[j] = pltpu.async_copy(hp.at[src_v.at[j]], rows[j], sems[j])
    for j in range(NCHUNK):
        b = j % _NBUF
        descs[j].wait()
        pltpu.sync_copy(rows[b], sh_agg.at[dst_v.at[j]], add=True)
        if j + _NBUF < NCHUNK:
            descs[j + _NBUF] = pltpu.async_copy(
                hp.at[src_v.at[j + _NBUF]], rows[b], sems[b])
    plsc.subcore_barrier()
    pltpu.sync_copy(sh_agg.at[pl.ds(s * RT, RT)], buf_v)
    pltpu.sync_copy(buf_v, agg_out.at[c, pl.ds(s * RT, RT)])


@functools.cache
def _sc_agg():
    return pl.kernel(
        _sc_agg_body,
        out_type=jax.ShapeDtypeStruct((NC, N, D_H), jnp.float32),
        mesh=_mesh(),
        scratch_types=[
            pltpu.VMEM((NCHUNK, CH), jnp.int32),
            pltpu.VMEM((NCHUNK, CH), jnp.int32),
            pltpu.VMEM((CH, D_H), jnp.float32),
            pltpu.VMEM((CH, D_H), jnp.float32),
            pltpu.VMEM((CH, D_H), jnp.float32),
            pltpu.VMEM((CH, D_H), jnp.float32),
            pltpu.VMEM((RT, D_H), jnp.float32),
            pltpu.VMEM_SHARED((N, D_H), jnp.float32),
            pltpu.SemaphoreType.DMA,
            pltpu.SemaphoreType.DMA,
            pltpu.SemaphoreType.DMA,
            pltpu.SemaphoreType.DMA,
        ],
        compiler_params=pltpu.CompilerParams(use_tc_tiling_on_sc=False),
    )


def _tc_a_body(xp_ref, w1b_ref, deg_ref, k_ref, hp_ref, dinv_ref):
    deg = deg_ref[...]
    dinv8 = lax.rsqrt(deg[0] + deg[1] + 1.0)
    dinv_p = jnp.dot(dinv8, k_ref[...], preferred_element_type=jnp.float32)
    h = jnp.dot(xp_ref[...], w1b_ref[...], preferred_element_type=jnp.float32)
    hp_ref[...] = h * dinv_p
    dinv_ref[...] = dinv_p


_tc_a = pl.pallas_call(
    _tc_a_body,
    out_shape=[
        jax.ShapeDtypeStruct((PK, 128), jnp.float32),
        jax.ShapeDtypeStruct((PK, 128), jnp.float32),
    ],
)


def _tc_b_body(agg_ref, hp_ref, dinv_ref, b1_ref, w2b_ref, out_ref):
    a3 = agg_ref[...]
    a = a3[0] + a3[1] + hp_ref[...]
    dinv = dinv_ref[...]
    z = jnp.maximum(a * dinv + b1_ref[...], 0.0)
    out_ref[...] = jnp.dot(
        z, w2b_ref[...], preferred_element_type=jnp.float32) * dinv


_tc_b = pl.pallas_call(
    _tc_b_body,
    out_shape=jax.ShapeDtypeStruct((PK, 128), jnp.float32),
)


def _tc_c_body(agg_ref, hp_ref, dinv_ref, b2_ref, out_ref):
    a3 = agg_ref[...]
    out_ref[...] = (a3[0] + a3[1] + hp_ref[...]) * dinv_ref[...] + b2_ref[...]


_tc_c = pl.pallas_call(
    _tc_c_body,
    out_shape=jax.ShapeDtypeStruct((PK, 128), jnp.float32),
)


@jax.jit
def kernel(x, edge_index, W1, b1, W2, b2):
    src3 = edge_index[0].reshape(NW, NCHUNK, CH)
    dst3 = edge_index[1].reshape(NW, NCHUNK, CH)
    zeros1 = jnp.zeros((ND,), jnp.float32)
    zeros2 = jnp.zeros((N, D_H), jnp.float32)
    eye8 = jnp.eye(8, dtype=jnp.float32)
    w1b = jnp.kron(eye8, W1)
    w2b = jnp.kron(eye8, W2)
    kmat = jnp.kron(eye8, jnp.ones((1, D_H), jnp.float32))
    xp = x.reshape(PK, 8 * D_IN)

    deg2 = _sc_deg()(dst3, zeros1)
    degp = deg2.reshape(NC, ND)[:, :N].reshape(NC, PK, 8)
    hp1_p, dinv_p = _tc_a(xp, w1b, degp, kmat)
    agg1 = _sc_agg()(hp1_p.reshape(N, D_H), src3, dst3, zeros2)
    hp2_p = _tc_b(agg1.reshape(NC, PK, 128), hp1_p, dinv_p,
                  jnp.tile(b1, 8).reshape(1, 128), w2b)
    agg2 = _sc_agg()(hp2_p.reshape(N, D_H), src3, dst3, zeros2)
    out_p = _tc_c(agg2.reshape(NC, PK, 128), hp2_p, dinv_p,
                  jnp.tile(b2, 8).reshape(1, 128))
    return out_p.reshape(N, D_H)


# speedup vs baseline: 81.0295x; 1.1189x over previous
import functools

import jax
import jax.numpy as jnp
from jax import lax
from jax.experimental import pallas as pl
from jax.experimental.pallas import tpu as pltpu
from jax.experimental.pallas import tpu_sc as plsc

N = 10000
E = 320000
D_IN = 128
D_H = 16

NC = 2
NS = 16
NW = NC * NS
CH = 125
NCHUNK = 80
EPW = NCHUNK * CH
RT = N // NS
ND = 10112
RTD = ND // NS
PK = N // 8


@functools.cache
def _mesh():
    return plsc.VectorSubcoreMesh(
        core_axis_name="c", subcore_axis_name="s",
        num_cores=NC, num_subcores=NS)


def _sc_deg_body(dst3, zeros1, deg_out, dst_v, ones_v, buf_v, sh_deg, sem):
    c = lax.axis_index("c")
    s = lax.axis_index("s")
    w = c * NS + s
    pltpu.sync_copy(zeros1.at[pl.ds(s * RTD, RTD)], buf_v)
    pltpu.sync_copy(buf_v, sh_deg.at[pl.ds(s * RTD, RTD)])
    for i in range(8):
        ones_v[pl.ds(i * 16, 16)] = jnp.ones((16,), jnp.float32)
    pltpu.sync_copy(dst3.at[w], dst_v)
    plsc.subcore_barrier()

    descs = [pltpu.async_copy(ones_v.at[pl.ds(0, CH)],
                              sh_deg.at[dst_v.at[j]], sem, add=True)
             for j in range(NCHUNK)]
    for d in descs:
        d.wait()
    plsc.subcore_barrier()
    off = pl.multiple_of(c * ND + s * RTD, 8)
    pltpu.sync_copy(sh_deg.at[pl.ds(s * RTD, RTD)], buf_v)
    pltpu.sync_copy(buf_v, deg_out.at[pl.ds(off, RTD)])


@functools.cache
def _sc_deg():
    return pl.kernel(
        _sc_deg_body,
        out_type=jax.ShapeDtypeStruct((NC * ND,), jnp.float32),
        mesh=_mesh(),
        scratch_types=[
            pltpu.VMEM((NCHUNK, CH), jnp.int32),
            pltpu.VMEM((128,), jnp.float32),
            pltpu.VMEM((RTD,), jnp.float32),
            pltpu.VMEM_SHARED((ND,), jnp.float32),
            pltpu.SemaphoreType.DMA,
        ],
        compiler_params=pltpu.CompilerParams(use_tc_tiling_on_sc=False),
    )


_NBUF = 8
_SLACK = 2


def _sc_agg_body(hp, src3, dst3, zeros2, agg_out, src_v, dst_v, rows, buf_v,
                 sh_agg, gsems, ssems):
    c = lax.axis_index("c")
    s = lax.axis_index("s")
    w = c * NS + s
    pltpu.sync_copy(zeros2.at[pl.ds(s * RT, RT)], buf_v)
    pltpu.sync_copy(buf_v, sh_agg.at[pl.ds(s * RT, RT)])
    pltpu.sync_copy(src3.at[w], src_v)
    pltpu.sync_copy(dst3.at[w], dst_v)
    plsc.subcore_barrier()

    gd = [None] * NCHUNK
    sd = [None] * NCHUNK
    for j in range(_NBUF):
        gd[j] = pltpu.async_copy(hp.at[src_v.at[j]], rows[j], gsems[j])
    for j in range(NCHUNK):
        b = j % _NBUF
        gd[j].wait()
        sd[j] = pltpu.async_copy(rows[b], sh_agg.at[dst_v.at[j]], ssems[b],
                                 add=True)
        jr = j - _SLACK
        if jr >= 0 and jr + _NBUF < NCHUNK:
            sd[jr].wait()
            gd[jr + _NBUF] = pltpu.async_copy(
                hp.at[src_v.at[jr + _NBUF]], rows[jr % _NBUF],
                gsems[jr % _NBUF])
    for j in range(NCHUNK - _NBUF, NCHUNK):
        sd[j].wait()
    plsc.subcore_barrier()
    pltpu.sync_copy(sh_agg.at[pl.ds(s * RT, RT)], buf_v)
    pltpu.sync_copy(buf_v, agg_out.at[c, pl.ds(s * RT, RT)])


@functools.cache
def _sc_agg():
    return pl.kernel(
        _sc_agg_body,
        out_type=jax.ShapeDtypeStruct((NC, N, D_H), jnp.float32),
        mesh=_mesh(),
        scratch_types=[
            pltpu.VMEM((NCHUNK, CH), jnp.int32),
            pltpu.VMEM((NCHUNK, CH), jnp.int32),
            [pltpu.VMEM((CH, D_H), jnp.float32) for _ in range(_NBUF)],
            pltpu.VMEM((RT, D_H), jnp.float32),
            pltpu.VMEM_SHARED((N, D_H), jnp.float32),
            [pltpu.SemaphoreType.DMA for _ in range(_NBUF)],
            [pltpu.SemaphoreType.DMA for _ in range(_NBUF)],
        ],
        compiler_params=pltpu.CompilerParams(use_tc_tiling_on_sc=False),
    )


def _tc_a_body(xp_ref, w1b_ref, deg_ref, k_ref, hp_ref, dinv_ref):
    deg = deg_ref[...]
    dinv8 = lax.rsqrt(deg[0] + deg[1] + 1.0)
    dinv_p = jnp.dot(dinv8, k_ref[...], preferred_element_type=jnp.float32)
    h = jnp.dot(xp_ref[...], w1b_ref[...], preferred_element_type=jnp.float32)
    hp_ref[...] = h * dinv_p
    dinv_ref[...] = dinv_p


_tc_a = pl.pallas_call(
    _tc_a_body,
    out_shape=[
        jax.ShapeDtypeStruct((PK, 128), jnp.float32),
        jax.ShapeDtypeStruct((PK, 128), jnp.float32),
    ],
)


def _tc_b_body(agg_ref, hp_ref, dinv_ref, b1_ref, w2b_ref, out_ref):
    a3 = agg_ref[...]
    a = a3[0] + a3[1] + hp_ref[...]
    dinv = dinv_ref[...]
    z = jnp.maximum(a * dinv + b1_ref[...], 0.0)
    out_ref[...] = jnp.dot(
        z, w2b_ref[...], preferred_element_type=jnp.float32) * dinv


_tc_b = pl.pallas_call(
    _tc_b_body,
    out_shape=jax.ShapeDtypeStruct((PK, 128), jnp.float32),
)


def _tc_c_body(agg_ref, hp_ref, dinv_ref, b2_ref, out_ref):
    a3 = agg_ref[...]
    out_ref[...] = (a3[0] + a3[1] + hp_ref[...]) * dinv_ref[...] + b2_ref[...]


_tc_c = pl.pallas_call(
    _tc_c_body,
    out_shape=jax.ShapeDtypeStruct((PK, 128), jnp.float32),
)


@jax.jit
def kernel(x, edge_index, W1, b1, W2, b2):
    src3 = edge_index[0].reshape(NW, NCHUNK, CH)
    dst3 = edge_index[1].reshape(NW, NCHUNK, CH)
    zeros1 = jnp.zeros((ND,), jnp.float32)
    zeros2 = jnp.zeros((N, D_H), jnp.float32)
    eye8 = jnp.eye(8, dtype=jnp.float32)
    w1b = jnp.kron(eye8, W1)
    w2b = jnp.kron(eye8, W2)
    kmat = jnp.kron(eye8, jnp.ones((1, D_H), jnp.float32))
    xp = x.reshape(PK, 8 * D_IN)

    deg2 = _sc_deg()(dst3, zeros1)
    degp = deg2.reshape(NC, ND)[:, :N].reshape(NC, PK, 8)
    hp1_p, dinv_p = _tc_a(xp, w1b, degp, kmat)
    agg1 = _sc_agg()(hp1_p.reshape(N, D_H), src3, dst3, zeros2)
    hp2_p = _tc_b(agg1.reshape(NC, PK, 128), hp1_p, dinv_p,
                  jnp.tile(b1, 8).reshape(1, 128), w2b)
    agg2 = _sc_agg()(hp2_p.reshape(N, D_H), src3, dst3, zeros2)
    out_p = _tc_c(agg2.reshape(NC, PK, 128), hp2_p, dinv_p,
                  jnp.tile(b2, 8).reshape(1, 128))
    return out_p.reshape(N, D_H)
